# baseline (speedup 1.0000x reference)
import functools

import jax
import jax.numpy as jnp
from jax import lax
from jax.experimental import pallas as pl
from jax.experimental.pallas import tpu as pltpu
from jax.experimental.pallas import tpu_sc as plsc

_N = 10000
_E = 320000
_D = 128
_RD = 16
_R = 32
_NB = 16
_IN_S = 2 * _D + _RD
_FOLD = 4
_NF = 2560
_NBINS = _NF * 128

_NUM_CORES = 2
_NUM_SUBCORES = 16
_NUM_WORKERS = _NUM_CORES * _NUM_SUBCORES
_ROW_W = 128
_ROWS = 79
_EPT_PAD = _ROWS * _ROW_W
_BASE_ROWS = 78
_EXTRA_FROM = _NUM_WORKERS - 4
_BPT = _NBINS // _NUM_SUBCORES
_CH = 10240
_DEPTH = 16


def _sc_count_body(edge_index_hbm, edge_type_hbm, out_hbm,
                   ei_v, typ_v, idx_v, ones_v, stage_a, stage_b, hist_sh,
                   sem, sem2, sem_r, sem_w):
    cid = lax.axis_index("c")
    sid = lax.axis_index("s")
    wid = sid * _NUM_CORES + cid
    base = (wid * _BASE_ROWS + jnp.maximum(wid - _EXTRA_FROM, 0)) * _ROW_W

    in1 = pltpu.make_async_copy(
        edge_index_hbm.at[:, pl.ds(base, _EPT_PAD)], ei_v, sem2)
    in1.start()
    in2 = pltpu.make_async_copy(
        edge_type_hbm.at[pl.ds(base, _EPT_PAD)], typ_v, sem2)
    in2.start()

    zvec = jnp.zeros((16,), jnp.float32)

    def _zero(i, carry):
        stage_a[pl.ds(i * 16, 16)] = zvec
        return carry

    lax.fori_loop(0, _CH // 16, _zero, 0)
    for q in range(_BPT // _CH):
        pltpu.make_async_copy(
            stage_a, hist_sh.at[pl.ds(sid * _BPT + q * _CH, _CH)],
            sem_r).start()

    ovec = jnp.ones((16,), jnp.float32)

    def _ones(i, carry):
        ones_v[pl.ds(i * 16, 16)] = ovec
        return carry

    lax.fori_loop(0, _ROW_W // 16, _ones, 0)

    in1.wait()
    in2.wait()

    for q in range(_BPT // _CH):
        pltpu.make_async_copy(
            stage_a, hist_sh.at[pl.ds(sid * _BPT, _CH)], sem_r).wait()

    plsc.subcore_barrier()

    def _compute_row(j):
        for k in range(_ROW_W // 16):
            off = j * _ROW_W + k * 16
            d = ei_v[1, pl.ds(off, 16)]
            t = typ_v[pl.ds(off, 16)]
            slot = lax.shift_right_logical(d * 13108, 25)
            idx_v[j, pl.ds(k * 16, 16)] = (
                d * 128 + t - slot * (128 * _NF - _R))

    def _fire(j):
        pltpu.make_async_copy(ones_v, hist_sh.at[idx_v.at[j]], sem).start()

    def _wait_one():
        pltpu.make_async_copy(ones_v, hist_sh.at[idx_v.at[0]], sem).wait()

    def _body_nowait(j, carry):
        _compute_row(j)
        _fire(j)
        return carry

    def _body(j, carry):
        _compute_row(j)
        _fire(j)
        _wait_one()
        return carry

    lax.fori_loop(0, _DEPTH, _body_nowait, 0)
    lax.fori_loop(_DEPTH, _ROWS - 1, _body, 0)

    _compute_row(_ROWS - 1)
    pad_vec = jnp.full((16,), _NBINS, jnp.int32)

    @pl.when(wid < _EXTRA_FROM)
    def _():
        def _pad(i, carry):
            idx_v[_ROWS - 1, pl.ds(i * 16, 16)] = pad_vec
            return carry

        lax.fori_loop(0, _ROW_W // 16, _pad, 0)

    _fire(_ROWS - 1)
    for _ in range(_DEPTH + 1):
        _wait_one()

    plsc.subcore_barrier()

    def _rd(c, buf, s):
        return pltpu.make_async_copy(
            hist_sh.at[pl.ds(sid * _BPT + c * _CH, _CH)], buf, s)

    def _wr(c, buf, s):
        return pltpu.make_async_copy(
            buf, out_hbm.at[pl.ds(cid * _NBINS + sid * _BPT + c * _CH, _CH)],
            s)

    _rd(0, stage_a, sem_r).start()
    _rd(1, stage_b, sem_w).start()
    _rd(0, stage_a, sem_r).wait()
    _wr(0, stage_a, sem_r).start()
    _rd(1, stage_b, sem_w).wait()
    _wr(1, stage_b, sem_w).start()
    _wr(0, stage_a, sem_r).wait()
    _wr(1, stage_b, sem_w).wait()


_sc_count = pl.kernel(
    _sc_count_body,
    out_type=jax.ShapeDtypeStruct((_NUM_CORES * _NBINS,), jnp.float32),
    mesh=plsc.VectorSubcoreMesh(core_axis_name="c", subcore_axis_name="s"),
    scratch_types=[
        pltpu.VMEM((2, _EPT_PAD), jnp.int32),
        pltpu.VMEM((_EPT_PAD,), jnp.int32),
        pltpu.VMEM((_ROWS, _ROW_W), jnp.int32),
        pltpu.VMEM((_ROW_W,), jnp.float32),
        pltpu.VMEM((_CH,), jnp.float32),
        pltpu.VMEM((_CH,), jnp.float32),
        pltpu.VMEM_SHARED((_NBINS + 16,), jnp.float32),
        pltpu.SemaphoreType.DMA,
        pltpu.SemaphoreType.DMA,
        pltpu.SemaphoreType.DMA,
        pltpu.SemaphoreType.DMA,
    ],
)


_BLKF = 2560


def _tc_attn_body(x_ref, cnt_ref, eemb_ref, br_ref, basr_ref,
                  coefr_ref, out_ref):
    k = pl.program_id(1)
    x = x_ref[...]
    basr = basr_ref[...][:, :, 0]
    rw = jnp.dot(coefr_ref[...], basr,
                 preferred_element_type=jnp.float32)
    wp = rw[:, :_D]
    wq = rw[:, _D:2 * _D]
    wr = rw[:, 2 * _D:]
    p = lax.dot_general(x, wp, (((1,), (1,)), ((), ())),
                        preferred_element_type=jnp.float32)
    q = lax.dot_general(x, wq, (((1,), (1,)), ((), ())),
                        preferred_element_type=jnp.float32)
    dvec = jnp.sum(eemb_ref[...] * wr, axis=1) + br_ref[...][:, 0]
    lane = lax.broadcasted_iota(jnp.int32, (128, _R), 0)
    col = lax.broadcasted_iota(jnp.int32, (128, _R), 1)
    sel = (lane == col + _R * k).astype(jnp.float32)
    c3 = cnt_ref[...]
    csum = c3[0] + c3[1]
    cnt = lax.dot_general(csum, sel, (((1,), (0,)), ((), ())),
                          preferred_element_type=jnp.float32)
    mask = cnt > 0.0
    score = p + jnp.where(mask, q, 0.0) + dvec[None, :]
    score = jnp.where(mask, score, jnp.float32(-10000000000.0))
    m = jnp.max(score, axis=1, keepdims=True)
    e = jnp.exp(score - m)
    z = jnp.sum(e, axis=1, keepdims=True)
    s = jnp.sum(jnp.where(mask, e, 0.0), axis=1, keepdims=True) / z
    out_ref[...] = x * (1.0 + s)


_tc_attn = pl.pallas_call(
    _tc_attn_body,
    grid=(_NF // _BLKF, _FOLD),
    in_specs=[
        pl.BlockSpec((_BLKF, _D), lambda i, k: (k * (_NF // _BLKF) + i, 0)),
        pl.BlockSpec((2, _BLKF, 128), lambda i, k: (0, i, 0)),
        pl.BlockSpec((_R, _RD), lambda i, k: (0, 0)),
        pl.BlockSpec((_R, 1), lambda i, k: (0, 0)),
        pl.BlockSpec((_NB, _IN_S, 1), lambda i, k: (0, 0, 0)),
        pl.BlockSpec((_R, _NB), lambda i, k: (0, 0)),
    ],
    out_specs=pl.BlockSpec((_BLKF, _D), lambda i, k: (k * (_NF // _BLKF) + i, 0)),
    out_shape=jax.ShapeDtypeStruct((_N, _D), jnp.float32),
)


def kernel(node_emb, edge_embeddings, b_s, b_r, bases_s, coeff_s, bases_r,
           coeff_r, edge_index, edge_type):
    del b_s, bases_s, coeff_s
    counts = _sc_count(edge_index, edge_type)
    counts3 = counts.reshape(_NUM_CORES, _NF, 128)
    return _tc_attn(node_emb, counts3, edge_embeddings, b_r,
                    bases_r, coeff_r)


# speedup vs baseline: 1.0006x; 1.0006x over previous
import functools

import jax
import jax.numpy as jnp
from jax import lax
from jax.experimental import pallas as pl
from jax.experimental.pallas import tpu as pltpu
from jax.experimental.pallas import tpu_sc as plsc

_N = 10000
_E = 320000
_D = 128
_RD = 16
_R = 32
_NB = 16
_IN_S = 2 * _D + _RD
_FOLD = 4
_NF = 2560
_NBINS = _NF * 128

_NUM_CORES = 2
_NUM_SUBCORES = 16
_NUM_WORKERS = _NUM_CORES * _NUM_SUBCORES
_ROW_W = 128
_ROWS = 79
_EPT_PAD = _ROWS * _ROW_W
_BASE_ROWS = 78
_EXTRA_FROM = _NUM_WORKERS - 4
_BPT = _NBINS // _NUM_SUBCORES
_CH = 10240
_DEPTH = 8


def _sc_count_body(edge_index_hbm, edge_type_hbm, out_hbm,
                   ei_v, typ_v, idx_v, ones_v, stage_a, stage_b, hist_sh,
                   sem, sem2, sem_r, sem_w):
    cid = lax.axis_index("c")
    sid = lax.axis_index("s")
    wid = sid * _NUM_CORES + cid
    base = (wid * _BASE_ROWS + jnp.maximum(wid - _EXTRA_FROM, 0)) * _ROW_W

    in1 = pltpu.make_async_copy(
        edge_index_hbm.at[:, pl.ds(base, _EPT_PAD)], ei_v, sem2)
    in1.start()
    in2 = pltpu.make_async_copy(
        edge_type_hbm.at[pl.ds(base, _EPT_PAD)], typ_v, sem2)
    in2.start()

    zvec = jnp.zeros((16,), jnp.float32)

    def _zero(i, carry):
        stage_a[pl.ds(i * 16, 16)] = zvec
        return carry

    lax.fori_loop(0, _CH // 16, _zero, 0)
    for q in range(_BPT // _CH):
        pltpu.make_async_copy(
            stage_a, hist_sh.at[pl.ds(sid * _BPT + q * _CH, _CH)],
            sem_r).start()

    ovec = jnp.ones((16,), jnp.float32)

    def _ones(i, carry):
        ones_v[pl.ds(i * 16, 16)] = ovec
        return carry

    lax.fori_loop(0, _ROW_W // 16, _ones, 0)

    in1.wait()
    in2.wait()

    for q in range(_BPT // _CH):
        pltpu.make_async_copy(
            stage_a, hist_sh.at[pl.ds(sid * _BPT, _CH)], sem_r).wait()

    plsc.subcore_barrier()

    def _compute_row(j):
        for k in range(_ROW_W // 16):
            off = j * _ROW_W + k * 16
            d = ei_v[1, pl.ds(off, 16)]
            t = typ_v[pl.ds(off, 16)]
            slot = lax.shift_right_logical(d * 13108, 25)
            idx_v[j, pl.ds(k * 16, 16)] = (
                d * 128 + t - slot * (128 * _NF - _R))

    def _fire(j):
        pltpu.make_async_copy(ones_v, hist_sh.at[idx_v.at[j]], sem).start()

    def _wait_one():
        pltpu.make_async_copy(ones_v, hist_sh.at[idx_v.at[0]], sem).wait()

    def _body_nowait(j, carry):
        _compute_row(j)
        _fire(j)
        return carry

    def _body(j, carry):
        _compute_row(j)
        _fire(j)
        _wait_one()
        return carry

    lax.fori_loop(0, _DEPTH, _body_nowait, 0)
    lax.fori_loop(_DEPTH, _ROWS - 1, _body, 0)

    _compute_row(_ROWS - 1)
    pad_vec = jnp.full((16,), _NBINS, jnp.int32)

    @pl.when(wid < _EXTRA_FROM)
    def _():
        def _pad(i, carry):
            idx_v[_ROWS - 1, pl.ds(i * 16, 16)] = pad_vec
            return carry

        lax.fori_loop(0, _ROW_W // 16, _pad, 0)

    _fire(_ROWS - 1)
    for _ in range(_DEPTH + 1):
        _wait_one()

    plsc.subcore_barrier()

    def _rd(c, buf, s):
        return pltpu.make_async_copy(
            hist_sh.at[pl.ds(sid * _BPT + c * _CH, _CH)], buf, s)

    def _wr(c, buf, s):
        return pltpu.make_async_copy(
            buf, out_hbm.at[pl.ds(cid * _NBINS + sid * _BPT + c * _CH, _CH)],
            s)

    _rd(0, stage_a, sem_r).start()
    _rd(1, stage_b, sem_w).start()
    _rd(0, stage_a, sem_r).wait()
    _wr(0, stage_a, sem_r).start()
    _rd(1, stage_b, sem_w).wait()
    _wr(1, stage_b, sem_w).start()
    _wr(0, stage_a, sem_r).wait()
    _wr(1, stage_b, sem_w).wait()


_sc_count = pl.kernel(
    _sc_count_body,
    out_type=jax.ShapeDtypeStruct((_NUM_CORES * _NBINS,), jnp.float32),
    mesh=plsc.VectorSubcoreMesh(core_axis_name="c", subcore_axis_name="s"),
    scratch_types=[
        pltpu.VMEM((2, _EPT_PAD), jnp.int32),
        pltpu.VMEM((_EPT_PAD,), jnp.int32),
        pltpu.VMEM((_ROWS, _ROW_W), jnp.int32),
        pltpu.VMEM((_ROW_W,), jnp.float32),
        pltpu.VMEM((_CH,), jnp.float32),
        pltpu.VMEM((_CH,), jnp.float32),
        pltpu.VMEM_SHARED((_NBINS + 16,), jnp.float32),
        pltpu.SemaphoreType.DMA,
        pltpu.SemaphoreType.DMA,
        pltpu.SemaphoreType.DMA,
        pltpu.SemaphoreType.DMA,
    ],
)


_BLKF = 2560


def _tc_attn_body(x_ref, cnt_ref, eemb_ref, br_ref, basr_ref,
                  coefr_ref, out_ref):
    k = pl.program_id(1)
    x = x_ref[...]
    basr = basr_ref[...][:, :, 0]
    rw = jnp.dot(coefr_ref[...], basr,
                 preferred_element_type=jnp.float32)
    wp = rw[:, :_D]
    wq = rw[:, _D:2 * _D]
    wr = rw[:, 2 * _D:]
    p = lax.dot_general(x, wp, (((1,), (1,)), ((), ())),
                        preferred_element_type=jnp.float32)
    q = lax.dot_general(x, wq, (((1,), (1,)), ((), ())),
                        preferred_element_type=jnp.float32)
    dvec = jnp.sum(eemb_ref[...] * wr, axis=1) + br_ref[...][:, 0]
    lane = lax.broadcasted_iota(jnp.int32, (128, _R), 0)
    col = lax.broadcasted_iota(jnp.int32, (128, _R), 1)
    sel = (lane == col + _R * k).astype(jnp.float32)
    c3 = cnt_ref[...]
    csum = c3[0] + c3[1]
    cnt = lax.dot_general(csum, sel, (((1,), (0,)), ((), ())),
                          preferred_element_type=jnp.float32)
    mask = cnt > 0.0
    score = p + jnp.where(mask, q, 0.0) + dvec[None, :]
    score = jnp.where(mask, score, jnp.float32(-10000000000.0))
    m = jnp.max(score, axis=1, keepdims=True)
    e = jnp.exp(score - m)
    z = jnp.sum(e, axis=1, keepdims=True)
    s = jnp.sum(jnp.where(mask, e, 0.0), axis=1, keepdims=True) / z
    out_ref[...] = x * (1.0 + s)


_tc_attn = pl.pallas_call(
    _tc_attn_body,
    grid=(_NF // _BLKF, _FOLD),
    in_specs=[
        pl.BlockSpec((_BLKF, _D), lambda i, k: (k * (_NF // _BLKF) + i, 0)),
        pl.BlockSpec((2, _BLKF, 128), lambda i, k: (0, i, 0)),
        pl.BlockSpec((_R, _RD), lambda i, k: (0, 0)),
        pl.BlockSpec((_R, 1), lambda i, k: (0, 0)),
        pl.BlockSpec((_NB, _IN_S, 1), lambda i, k: (0, 0, 0)),
        pl.BlockSpec((_R, _NB), lambda i, k: (0, 0)),
    ],
    out_specs=pl.BlockSpec((_BLKF, _D), lambda i, k: (k * (_NF // _BLKF) + i, 0)),
    out_shape=jax.ShapeDtypeStruct((_N, _D), jnp.float32),
)


def kernel(node_emb, edge_embeddings, b_s, b_r, bases_s, coeff_s, bases_r,
           coeff_r, edge_index, edge_type):
    del b_s, bases_s, coeff_s
    counts = _sc_count(edge_index, edge_type)
    counts3 = counts.reshape(_NUM_CORES, _NF, 128)
    return _tc_attn(node_emb, counts3, edge_embeddings, b_r,
                    bases_r, coeff_r)
